# baseline (speedup 1.0000x reference)
import functools

import jax
import jax.numpy as jnp
from jax import lax
from jax.experimental import pallas as pl
from jax.experimental.pallas import tpu as pltpu
from jax.experimental.pallas import tpu_sc as plsc

_VOCAB = 18440
_EMBED = 100
_OUT = 2
_B = 4096
_L = 200

_NW = 32
_RPW = _B // _NW
_GPW = _RPW // 16
_LW = _L // 2
_TPJ = 104


def _prep_body(w_ref, t_ref, tok_ref, tw_ref, tokw_ref):
    tw_t = lax.dot_general(
        w_ref[...], t_ref[...],
        (((0,), (1,)), ((), ())),
        preferred_element_type=jnp.float32,
    ) * (1.0 / _L)
    bits = lax.bitcast_convert_type(tw_t.astype(jnp.bfloat16), jnp.uint16)
    packed = bits[0].astype(jnp.uint32) | (bits[1].astype(jnp.uint32) * 65536)
    tw_ref[...] = packed.astype(jnp.int32)

    tok = tok_ref[...]
    words = tok[:, : _LW] | (tok[:, _LW :] * 65536)
    w3 = lax.transpose(words.reshape(_NW, _RPW, _LW), (0, 2, 1))
    tokw_ref[:, : _LW, :] = w3


def _prep(table, W, text_token):
    return pl.pallas_call(
        _prep_body,
        out_shape=(
            jax.ShapeDtypeStruct((_VOCAB,), jnp.int32),
            jax.ShapeDtypeStruct((_NW, _TPJ, _RPW), jnp.int32),
        ),
    )(W, table, text_token)


@functools.partial(
    pl.kernel,
    out_type=jax.ShapeDtypeStruct((_NW, _OUT, _RPW), jnp.float32),
    mesh=plsc.VectorSubcoreMesh(core_axis_name="c", subcore_axis_name="s"),
    compiler_params=pltpu.CompilerParams(needs_layout_passes=False),
    scratch_types=[
        pltpu.VMEM((_TPJ * _RPW,), jnp.int32),
        pltpu.VMEM((_VOCAB,), jnp.int32),
        pltpu.VMEM((_OUT * 16,), jnp.float32),
        pltpu.VMEM((_OUT, _RPW), jnp.float32),
    ],
)
def _sc_pool(tok_hbm, tw_hbm, bias_hbm, out_hbm, tok_v, tw_v, bias_v, out_v):
    wid = lax.axis_index("s") * 2 + lax.axis_index("c")
    nword = _TPJ * _RPW
    pltpu.sync_copy(tok_hbm.at[pl.ds(wid * nword, nword)], tok_v)
    pltpu.sync_copy(tw_hbm, tw_v)
    pltpu.sync_copy(bias_hbm, bias_v)
    bv0 = bias_v[pl.ds(0, 16)]
    bv1 = bias_v[pl.ds(16, 16)]
    lo_mask = jnp.full((16,), 0xFFFF, jnp.int32)
    hi_mask = jnp.full((16,), 0xFFFF0000, jnp.uint32).astype(jnp.int32)

    def unpack_pair(w):
        v0 = lax.bitcast_convert_type(lax.shift_left(w, 16), jnp.float32)
        v1 = lax.bitcast_convert_type(lax.bitwise_and(w, hi_mask), jnp.float32)
        return v0, v1

    def body(jj, carry):
        acc = list(carry)
        for u in range(2):
            for g in range(_GPW):
                a0, a1 = acc[2 * g], acc[2 * g + 1]
                tokw = tok_v[pl.ds((jj * 2 + u) * _RPW + g * 16, 16)]
                t_lo = lax.bitwise_and(tokw, lo_mask)
                t_hi = lax.shift_right_logical(tokw, 16)
                w_lo = plsc.load_gather(tw_v, [t_lo])
                w_hi = plsc.load_gather(tw_v, [t_hi])
                v0l, v1l = unpack_pair(w_lo)
                v0h, v1h = unpack_pair(w_hi)
                acc[2 * g] = a0 + (v0l + v0h)
                acc[2 * g + 1] = a1 + (v1l + v1h)
        return tuple(acc)

    zero = jnp.zeros((16,), jnp.float32)
    accs = lax.fori_loop(0, _LW // 2, body, (zero,) * (2 * _GPW))
    for g in range(_GPW):
        out_v[0, pl.ds(g * 16, 16)] = accs[2 * g] + bv0
        out_v[1, pl.ds(g * 16, 16)] = accs[2 * g + 1] + bv1

    pltpu.sync_copy(out_v, out_hbm.at[wid])


def kernel(text_token, table, W, b):
    tw_pack, tok_words = _prep(table, W, text_token)
    bias16 = jnp.broadcast_to(b[:, None], (_OUT, 16)).reshape(-1)
    out = _sc_pool(tok_words.reshape(-1), tw_pack, bias16)
    return jnp.transpose(out, (0, 2, 1)).reshape(_B, _OUT)


# speedup vs baseline: 1.0109x; 1.0109x over previous
import functools

import jax
import jax.numpy as jnp
from jax import lax
from jax.experimental import pallas as pl
from jax.experimental.pallas import tpu as pltpu
from jax.experimental.pallas import tpu_sc as plsc

_VOCAB = 18440
_EMBED = 100
_OUT = 2
_B = 4096
_L = 200

_NW = 32
_RPW = _B // _NW
_GPW = _RPW // 16
_LW = _L // 2
_TPJ = 104


def _prep_body(w_ref, t_ref, tok_ref, tw_ref, tokw_ref):
    tw_t = lax.dot_general(
        w_ref[...], t_ref[...],
        (((0,), (1,)), ((), ())),
        preferred_element_type=jnp.float32,
    ) * (1.0 / _L)
    bits = lax.bitcast_convert_type(tw_t.astype(jnp.bfloat16), jnp.uint16)
    packed = bits[0].astype(jnp.uint32) | (bits[1].astype(jnp.uint32) * 65536)
    tw_ref[...] = packed.astype(jnp.int32)

    tok = tok_ref[...]
    words = tok[:, : _LW] | (tok[:, _LW :] * 65536)
    w3 = lax.transpose(words.reshape(_NW, _RPW, _LW), (0, 2, 1))
    tokw_ref[:, : _LW, :] = w3


def _prep(table, W, text_token):
    return pl.pallas_call(
        _prep_body,
        out_shape=(
            jax.ShapeDtypeStruct((_VOCAB,), jnp.int32),
            jax.ShapeDtypeStruct((_NW, _TPJ, _RPW), jnp.int32),
        ),
    )(W, table, text_token)


@functools.partial(
    pl.kernel,
    out_type=jax.ShapeDtypeStruct((_NW, _OUT, _RPW), jnp.float32),
    mesh=plsc.VectorSubcoreMesh(core_axis_name="c", subcore_axis_name="s"),
    compiler_params=pltpu.CompilerParams(needs_layout_passes=False),
    scratch_types=[
        pltpu.VMEM((_TPJ * _RPW,), jnp.int32),
        pltpu.VMEM((_VOCAB,), jnp.int32),
        pltpu.VMEM((_OUT * 16,), jnp.float32),
        pltpu.VMEM((_OUT, _RPW), jnp.float32),
    ],
)
def _sc_pool(tok_hbm, tw_hbm, bias_hbm, out_hbm, tok_v, tw_v, bias_v, out_v):
    wid = lax.axis_index("s") * 2 + lax.axis_index("c")
    nword = _TPJ * _RPW
    pltpu.sync_copy(tok_hbm.at[pl.ds(wid * nword, nword)], tok_v)
    pltpu.sync_copy(tw_hbm, tw_v)
    pltpu.sync_copy(bias_hbm, bias_v)
    bv0 = bias_v[pl.ds(0, 16)]
    bv1 = bias_v[pl.ds(16, 16)]
    lo_mask = jnp.full((16,), 0xFFFF, jnp.int32)
    hi_mask = jnp.full((16,), 0xFFFF0000, jnp.uint32).astype(jnp.int32)

    def unpack_pair(w):
        v0 = lax.bitcast_convert_type(lax.shift_left(w, 16), jnp.float32)
        v1 = lax.bitcast_convert_type(lax.bitwise_and(w, hi_mask), jnp.float32)
        return v0, v1

    def body(j, carry):
        new = []
        for g in range(_GPW):
            a0, a1 = carry[2 * g], carry[2 * g + 1]
            tokw = tok_v[pl.ds(j * _RPW + g * 16, 16)]
            t_lo = lax.bitwise_and(tokw, lo_mask)
            t_hi = lax.shift_right_logical(tokw, 16)
            w_lo = plsc.load_gather(tw_v, [t_lo])
            w_hi = plsc.load_gather(tw_v, [t_hi])
            v0l, v1l = unpack_pair(w_lo)
            v0h, v1h = unpack_pair(w_hi)
            new.append(a0 + (v0l + v0h))
            new.append(a1 + (v1l + v1h))
        return tuple(new)

    zero = jnp.zeros((16,), jnp.float32)
    accs = lax.fori_loop(0, _LW, body, (zero,) * (2 * _GPW))
    for g in range(_GPW):
        out_v[0, pl.ds(g * 16, 16)] = accs[2 * g] + bv0
        out_v[1, pl.ds(g * 16, 16)] = accs[2 * g + 1] + bv1

    pltpu.sync_copy(out_v, out_hbm.at[wid])


def kernel(text_token, table, W, b):
    tw_pack, tok_words = _prep(table, W, text_token)
    bias16 = jnp.broadcast_to(b[:, None], (_OUT, 16)).reshape(-1)
    out = _sc_pool(tok_words.reshape(-1), tw_pack, bias16)
    return jnp.transpose(out, (0, 2, 1)).reshape(_B, _OUT)


# speedup vs baseline: 1.0111x; 1.0001x over previous
import functools

import jax
import jax.numpy as jnp
from jax import lax
from jax.experimental import pallas as pl
from jax.experimental.pallas import tpu as pltpu
from jax.experimental.pallas import tpu_sc as plsc

_VOCAB = 18440
_EMBED = 100
_OUT = 2
_B = 4096
_L = 200

_NW = 32
_RPW = _B // _NW
_GPW = _RPW // 16
_LW = _L // 2
_TPJ = 104


def _prep_body(w_ref, t_ref, tok_ref, tw_ref, tokw_ref):
    tw_t = lax.dot_general(
        w_ref[...], t_ref[...],
        (((0,), (1,)), ((), ())),
        preferred_element_type=jnp.float32,
    ) * (1.0 / _L)
    bits = lax.bitcast_convert_type(tw_t.astype(jnp.bfloat16), jnp.uint16)
    packed = bits[0].astype(jnp.uint32) | (bits[1].astype(jnp.uint32) * 65536)
    tw_ref[...] = packed.astype(jnp.int32)

    tok = tok_ref[...]
    words = tok[:, : _LW] | (tok[:, _LW :] * 65536)
    w3 = lax.transpose(words.reshape(_NW, _RPW, _LW), (0, 2, 1))
    tokw_ref[:, : _LW, :] = w3


def _prep(table, W, text_token):
    return pl.pallas_call(
        _prep_body,
        out_shape=(
            jax.ShapeDtypeStruct((_VOCAB,), jnp.int32),
            jax.ShapeDtypeStruct((_NW, _TPJ, _RPW), jnp.int32),
        ),
    )(W, table, text_token)


@functools.partial(
    pl.kernel,
    out_type=jax.ShapeDtypeStruct((_NW, _OUT, _RPW), jnp.float32),
    mesh=plsc.VectorSubcoreMesh(core_axis_name="c", subcore_axis_name="s"),
    compiler_params=pltpu.CompilerParams(needs_layout_passes=False),
    scratch_types=[
        pltpu.VMEM((_TPJ * _RPW,), jnp.int32),
        pltpu.VMEM((_VOCAB,), jnp.int32),
        pltpu.VMEM((_OUT * 16,), jnp.float32),
        pltpu.VMEM((_OUT, _RPW), jnp.float32),
    ],
)
def _sc_pool(tok_hbm, tw_hbm, bias_hbm, out_hbm, tok_v, tw_v, bias_v, out_v):
    wid = lax.axis_index("s") * 2 + lax.axis_index("c")
    nword = _TPJ * _RPW
    pltpu.sync_copy(tok_hbm.at[pl.ds(wid * nword, nword)], tok_v)
    pltpu.sync_copy(tw_hbm, tw_v)
    pltpu.sync_copy(bias_hbm, bias_v)
    bv0 = bias_v[pl.ds(0, 16)]
    bv1 = bias_v[pl.ds(16, 16)]
    lo_mask = jnp.full((16,), 0xFFFF, jnp.int32)
    hi_mask = jnp.full((16,), 0xFFFF0000, jnp.uint32).astype(jnp.int32)

    def unpack_pair(w):
        v0 = lax.bitcast_convert_type(lax.shift_left(w, 16), jnp.float32)
        v1 = lax.bitcast_convert_type(lax.bitwise_and(w, hi_mask), jnp.float32)
        return v0, v1

    zero = jnp.zeros((16,), jnp.float32)

    @plsc.parallel_loop(0, _LW, carry=(zero,) * (2 * _GPW))
    def accs(j, carry):
        new = []
        for g in range(_GPW):
            a0, a1 = carry[2 * g], carry[2 * g + 1]
            tokw = tok_v[pl.ds(j * _RPW + g * 16, 16)]
            t_lo = lax.bitwise_and(tokw, lo_mask)
            t_hi = lax.shift_right_logical(tokw, 16)
            w_lo = plsc.load_gather(tw_v, [t_lo])
            w_hi = plsc.load_gather(tw_v, [t_hi])
            v0l, v1l = unpack_pair(w_lo)
            v0h, v1h = unpack_pair(w_hi)
            new.append(a0 + (v0l + v0h))
            new.append(a1 + (v1l + v1h))
        return tuple(new)
    for g in range(_GPW):
        out_v[0, pl.ds(g * 16, 16)] = accs[2 * g] + bv0
        out_v[1, pl.ds(g * 16, 16)] = accs[2 * g + 1] + bv1

    pltpu.sync_copy(out_v, out_hbm.at[wid])


def kernel(text_token, table, W, b):
    tw_pack, tok_words = _prep(table, W, text_token)
    bias16 = jnp.broadcast_to(b[:, None], (_OUT, 16)).reshape(-1)
    out = _sc_pool(tok_words.reshape(-1), tw_pack, bias16)
    return jnp.transpose(out, (0, 2, 1)).reshape(_B, _OUT)


# speedup vs baseline: 1.3818x; 1.3666x over previous
import functools

import jax
import jax.numpy as jnp
from jax import lax
from jax.experimental import pallas as pl
from jax.experimental.pallas import tpu as pltpu
from jax.experimental.pallas import tpu_sc as plsc

_VOCAB = 18440
_EMBED = 100
_OUT = 2
_B = 4096
_L = 200

_NW = 32
_RPW = _B // _NW
_GPW = _RPW // 16
_LW = _L // 2
_TPJ = 104


def _prep_body(w_ref, tt_ref, tokt_ref, tw_ref, tokw_ref):
    tw_t = lax.dot_general(
        w_ref[...], tt_ref[...],
        (((0,), (0,)), ((), ())),
        preferred_element_type=jnp.float32,
    ) * (1.0 / _L)
    bits = lax.bitcast_convert_type(tw_t.astype(jnp.bfloat16), jnp.uint16)
    packed = bits[0].astype(jnp.uint32) | (bits[1].astype(jnp.uint32) * 65536)
    tw_ref[...] = packed.astype(jnp.int32)

    tokt = tokt_ref[...]
    words_t = tokt[: _LW, :] | (tokt[_LW :, :] * 65536)
    w3 = lax.transpose(words_t.reshape(_LW, _NW, _RPW), (1, 0, 2))
    tokw_ref[:, : _LW, :] = w3


def _prep(table_t, W, text_token_t):
    return pl.pallas_call(
        _prep_body,
        out_shape=(
            jax.ShapeDtypeStruct((_VOCAB,), jnp.int32),
            jax.ShapeDtypeStruct((_NW, _TPJ, _RPW), jnp.int32),
        ),
    )(W, table_t, text_token_t)


@functools.partial(
    pl.kernel,
    out_type=jax.ShapeDtypeStruct((_NW, _OUT, _RPW), jnp.float32),
    mesh=plsc.VectorSubcoreMesh(core_axis_name="c", subcore_axis_name="s"),
    compiler_params=pltpu.CompilerParams(needs_layout_passes=False),
    scratch_types=[
        pltpu.VMEM((_TPJ * _RPW,), jnp.int32),
        pltpu.VMEM((_VOCAB,), jnp.int32),
        pltpu.VMEM((_OUT * 16,), jnp.float32),
        pltpu.VMEM((_OUT, _RPW), jnp.float32),
    ],
)
def _sc_pool(tok_hbm, tw_hbm, bias_hbm, out_hbm, tok_v, tw_v, bias_v, out_v):
    wid = lax.axis_index("s") * 2 + lax.axis_index("c")
    nword = _TPJ * _RPW
    pltpu.sync_copy(tok_hbm.at[pl.ds(wid * nword, nword)], tok_v)
    pltpu.sync_copy(tw_hbm, tw_v)
    pltpu.sync_copy(bias_hbm, bias_v)
    bv0 = bias_v[pl.ds(0, 16)]
    bv1 = bias_v[pl.ds(16, 16)]
    lo_mask = jnp.full((16,), 0xFFFF, jnp.int32)
    hi_mask = jnp.full((16,), 0xFFFF0000, jnp.uint32).astype(jnp.int32)

    def unpack_pair(w):
        v0 = lax.bitcast_convert_type(lax.shift_left(w, 16), jnp.float32)
        v1 = lax.bitcast_convert_type(lax.bitwise_and(w, hi_mask), jnp.float32)
        return v0, v1

    zero = jnp.zeros((16,), jnp.float32)

    @plsc.parallel_loop(0, _LW, carry=(zero,) * (2 * _GPW))
    def accs(j, carry):
        new = []
        for g in range(_GPW):
            a0, a1 = carry[2 * g], carry[2 * g + 1]
            tokw = tok_v[pl.ds(j * _RPW + g * 16, 16)]
            t_lo = lax.bitwise_and(tokw, lo_mask)
            t_hi = lax.shift_right_logical(tokw, 16)
            w_lo = plsc.load_gather(tw_v, [t_lo])
            w_hi = plsc.load_gather(tw_v, [t_hi])
            v0l, v1l = unpack_pair(w_lo)
            v0h, v1h = unpack_pair(w_hi)
            new.append(a0 + (v0l + v0h))
            new.append(a1 + (v1l + v1h))
        return tuple(new)
    for g in range(_GPW):
        out_v[0, pl.ds(g * 16, 16)] = accs[2 * g] + bv0
        out_v[1, pl.ds(g * 16, 16)] = accs[2 * g + 1] + bv1

    pltpu.sync_copy(out_v, out_hbm.at[wid])


def kernel(text_token, table, W, b):
    tw_pack, tok_words = _prep(table.T, W, text_token.T)
    bias16 = jnp.broadcast_to(b[:, None], (_OUT, 16)).reshape(-1)
    out = _sc_pool(tok_words.reshape(-1), tw_pack, bias16)
    return jnp.transpose(out, (0, 2, 1)).reshape(_B, _OUT)
